# baseline (speedup 1.0000x reference)
import functools

import jax
import jax.numpy as jnp
from jax import lax
from jax.experimental import pallas as pl
from jax.experimental.pallas import tpu as pltpu
from jax.experimental.pallas import tpu_sc as plsc

ROWS, COLS = 4096, 50
EMBED = 64
VOCAB = 100000
NC, NS = 2, 16
NW = NC * NS
EPW = EMBED // NW
LANES = 16
NVEC = ROWS // LANES
NBC = 4
NBE = 3
STEP = 12
MAIN = COLS - COLS % STEP


def _build():
    mesh = plsc.VectorSubcoreMesh(core_axis_name="c", subcore_axis_name="s")

    @functools.partial(
        pl.kernel,
        mesh=mesh,
        out_type=jax.ShapeDtypeStruct((COLS, EMBED, ROWS), jnp.float32),
        scratch_types=[
            pltpu.VMEM((1, VOCAB), jnp.float32),
            pltpu.VMEM((NBC, 1, ROWS), jnp.int32),
            pltpu.VMEM((NBE, 1, 1, ROWS), jnp.float32),
            pltpu.SemaphoreType.DMA,
            pltpu.SemaphoreType.DMA((NBC,)),
            pltpu.SemaphoreType.DMA((NBE,)),
        ],
        compiler_params=pltpu.CompilerParams(use_tc_tiling_on_sc=True,
                                             needs_layout_passes=False),
    )
    def gather_kernel(sidt_hbm, tablet_hbm, out_hbm,
                      plane_v, col_v, cell_v, tsem, csem, osem):
        wid = lax.axis_index("s") * NC + lax.axis_index("c")

        def col_start(s, b):
            pltpu.async_copy(sidt_hbm.at[pl.ds(s, 1), :], col_v.at[b],
                             csem.at[b])

        def col_wait(b):
            pltpu.make_async_copy(sidt_hbm.at[pl.ds(0, 1), :], col_v.at[b],
                                  csem.at[b]).wait()

        def cell_start(e, s, b):
            pltpu.async_copy(cell_v.at[b],
                             out_hbm.at[pl.ds(s, 1), pl.ds(e, 1), :],
                             osem.at[b])

        def cell_wait(b):
            pltpu.make_async_copy(cell_v.at[b],
                                  out_hbm.at[pl.ds(0, 1), pl.ds(0, 1), :],
                                  osem.at[b]).wait()

        def compute(s, bc, be, e):
            @plsc.parallel_loop(0, NVEC, unroll=32)
            def _vec(k):
                idx = col_v.at[bc, 0][pl.ds(k * LANES, LANES)]
                vals = plsc.load_gather(plane_v.at[0], [idx])
                cell_v.at[be, 0, 0][pl.ds(k * LANES, LANES)] = vals

            cell_start(e, s, be)

        for ei in range(EPW):
            e = wid * EPW + ei
            pltpu.async_copy(tablet_hbm.at[pl.ds(e, 1), :], plane_v, tsem)
            pltpu.make_async_copy(tablet_hbm.at[pl.ds(0, 1), :], plane_v,
                                  tsem).wait()
            for b in range(NBC):
                col_start(b, b)

            @pl.loop(0, MAIN, step=STEP)
            def _sblk(s0):
                for j in range(STEP):
                    s = s0 + j
                    bc = j % NBC
                    be = j % NBE
                    col_wait(bc)

                    @pl.when(s >= NBE)
                    def _():
                        cell_wait(be)

                    compute(s, bc, be, e)

                    @pl.when(s + NBC < COLS)
                    def _():
                        col_start(s + NBC, bc)

            for s in range(MAIN, COLS):
                bc = s % NBC
                be = s % NBE
                col_wait(bc)
                cell_wait(be)
                compute(s, bc, be, e)

            for be in range(NBE):
                cell_wait(be)

    return gather_kernel


_GATHER = _build()


def kernel(sid, table):
    tablet = table.astype(jnp.float32).T
    sidt = sid.astype(jnp.int32).T
    out3 = _GATHER(sidt, tablet)
    return out3.transpose(2, 0, 1)


# speedup vs baseline: 1.0495x; 1.0495x over previous
import functools

import jax
import jax.numpy as jnp
from jax import lax
from jax.experimental import pallas as pl
from jax.experimental.pallas import tpu as pltpu
from jax.experimental.pallas import tpu_sc as plsc

ROWS, COLS = 4096, 50
EMBED = 64
VOCAB = 100000
NC, NS = 2, 16
NW = NC * NS
EPW = EMBED // NW
LANES = 16
NVEC = ROWS // LANES
NBC = 4
NBE = 3
STEP = 12
MAIN = COLS - COLS % STEP


def _build():
    mesh = plsc.VectorSubcoreMesh(core_axis_name="c", subcore_axis_name="s")

    @functools.partial(
        pl.kernel,
        mesh=mesh,
        out_type=jax.ShapeDtypeStruct((COLS, EMBED, ROWS), jnp.float32),
        scratch_types=[
            pltpu.VMEM((1, VOCAB), jnp.float32),
            pltpu.VMEM((NBC, 1, ROWS), jnp.int32),
            pltpu.VMEM((NBE, 1, 1, ROWS), jnp.float32),
            pltpu.SemaphoreType.DMA,
            pltpu.SemaphoreType.DMA((NBC,)),
            pltpu.SemaphoreType.DMA((NBE,)),
        ],
        compiler_params=pltpu.CompilerParams(use_tc_tiling_on_sc=True,
                                             needs_layout_passes=False),
    )
    def gather_kernel(sidt_hbm, tablet_hbm, out_hbm,
                      plane_v, col_v, cell_v, tsem, csem, osem):
        wid = lax.axis_index("s") * NC + lax.axis_index("c")

        def col_start(s, b):
            pltpu.async_copy(sidt_hbm.at[pl.ds(s, 1), :], col_v.at[b],
                             csem.at[b])

        def col_wait(b):
            pltpu.make_async_copy(sidt_hbm.at[pl.ds(0, 1), :], col_v.at[b],
                                  csem.at[b]).wait()

        def cell_start(e, s, b):
            pltpu.async_copy(cell_v.at[b],
                             out_hbm.at[pl.ds(s, 1), pl.ds(e, 1), :],
                             osem.at[b])

        def cell_wait(b):
            pltpu.make_async_copy(cell_v.at[b],
                                  out_hbm.at[pl.ds(0, 1), pl.ds(0, 1), :],
                                  osem.at[b]).wait()

        def compute(s, bc, be, e):
            @plsc.parallel_loop(0, NVEC, unroll=16)
            def _vec(k):
                idx = col_v.at[bc, 0][pl.ds(k * LANES, LANES)]
                vals = plsc.load_gather(plane_v.at[0], [idx])
                cell_v.at[be, 0, 0][pl.ds(k * LANES, LANES)] = vals

            cell_start(e, s, be)

        for ei in range(EPW):
            e = wid * EPW + ei
            pltpu.async_copy(tablet_hbm.at[pl.ds(e, 1), :], plane_v, tsem)
            pltpu.make_async_copy(tablet_hbm.at[pl.ds(0, 1), :], plane_v,
                                  tsem).wait()
            for b in range(NBC):
                col_start(b, b)

            @pl.loop(0, MAIN, step=STEP)
            def _sblk(s0):
                for j in range(STEP):
                    s = s0 + j
                    bc = j % NBC
                    be = j % NBE
                    col_wait(bc)

                    @pl.when(s >= NBE)
                    def _():
                        cell_wait(be)

                    compute(s, bc, be, e)

                    @pl.when(s + NBC < COLS)
                    def _():
                        col_start(s + NBC, bc)

            for s in range(MAIN, COLS):
                bc = s % NBC
                be = s % NBE
                col_wait(bc)
                cell_wait(be)
                compute(s, bc, be, e)

            for be in range(NBE):
                cell_wait(be)

    return gather_kernel


_GATHER = _build()


def kernel(sid, table):
    tablet = table.astype(jnp.float32).T
    sidt = sid.astype(jnp.int32).T
    out3 = _GATHER(sidt, tablet)
    return out3.transpose(2, 0, 1)
